# baseline (speedup 1.0000x reference)
import jax
import jax.numpy as jnp
from jax import lax
from jax.experimental import pallas as pl
from jax.experimental.pallas import tpu as pltpu
from jax.experimental.pallas import tpu_sc as plsc

HIDDEN = 64
NUM_CORES = 2
NUM_SUBCORES = 16
NUM_WORKERS = NUM_CORES * NUM_SUBCORES
CHUNK = 1024


def _gather_body(idx_hbm, table_hbm, out_hbm, idx_v, rows_v, sem):
    wid = lax.axis_index("s") * NUM_CORES + lax.axis_index("c")
    b_per_w = idx_hbm.shape[0] // NUM_WORKERS
    n_chunks = b_per_w // CHUNK

    def body(i, carry):
        base = wid * b_per_w + i * CHUNK
        pltpu.sync_copy(idx_hbm.at[pl.ds(base, CHUNK)], idx_v)
        pltpu.async_copy(table_hbm.at[idx_v], rows_v, sem).wait()
        pltpu.sync_copy(rows_v, out_hbm.at[pl.ds(base, CHUNK)])
        return carry

    lax.fori_loop(0, n_chunks, body, 0)


def kernel(inputs, weight):
    batch, length = inputs.shape
    total = batch * length
    flat_idx = inputs.reshape(total).astype(jnp.int32)
    mesh = plsc.VectorSubcoreMesh(core_axis_name="c", subcore_axis_name="s")
    k = pl.kernel(
        _gather_body,
        mesh=mesh,
        out_type=jax.ShapeDtypeStruct((total, HIDDEN), jnp.float32),
        scratch_types=[
            pltpu.VMEM((CHUNK,), jnp.int32),
            pltpu.VMEM((CHUNK, HIDDEN), jnp.float32),
            pltpu.SemaphoreType.DMA,
        ],
        compiler_params=pltpu.CompilerParams(use_tc_tiling_on_sc=False),
    )
    out = k(flat_idx, weight)
    return out.reshape(batch, length, HIDDEN)


# speedup vs baseline: 1.0205x; 1.0205x over previous
import jax
import jax.numpy as jnp
from jax import lax
from jax.experimental import pallas as pl
from jax.experimental.pallas import tpu as pltpu
from jax.experimental.pallas import tpu_sc as plsc

HIDDEN = 64
NUM_CORES = 2
NUM_SUBCORES = 16
NUM_WORKERS = NUM_CORES * NUM_SUBCORES
CHUNK = 400
NBUF = 4


def _gather_body(idx_hbm, table_hbm, out_hbm, *refs):
    idx_bufs = refs[0:NBUF]
    row_bufs = refs[NBUF:2 * NBUF]
    g_sems = refs[2 * NBUF:3 * NBUF]
    o_sems = refs[3 * NBUF:4 * NBUF]

    wid = lax.axis_index("s") * NUM_CORES + lax.axis_index("c")
    b_per_w = idx_hbm.shape[0] // NUM_WORKERS
    n_chunks = b_per_w // NBUF // CHUNK * NBUF
    w_base = wid * b_per_w

    def start_gather(c, b):
        pltpu.sync_copy(idx_hbm.at[pl.ds(w_base + c * CHUNK, CHUNK)],
                        idx_bufs[b])
        pltpu.async_copy(table_hbm.at[idx_bufs[b]], row_bufs[b], g_sems[b])

    def wait_gather(b):
        pltpu.make_async_copy(table_hbm.at[idx_bufs[b]], row_bufs[b],
                              g_sems[b]).wait()

    def start_write(c, b):
        pltpu.async_copy(row_bufs[b],
                         out_hbm.at[pl.ds(w_base + c * CHUNK, CHUNK)],
                         o_sems[b])

    def wait_write(c, b):
        pltpu.make_async_copy(row_bufs[b],
                              out_hbm.at[pl.ds(w_base + c * CHUNK, CHUNK)],
                              o_sems[b]).wait()

    start_gather(0, 0)
    start_gather(1, 1)

    def outer(j, carry):
        for b in range(NBUF):
            c = j * NBUF + b
            b2 = (b + 2) % NBUF

            @pl.when(c + 2 < n_chunks)
            def _prefetch():
                @pl.when(c >= 2)
                def _drain():
                    wait_write(c - 2, b2)
                start_gather(c + 2, b2)

            wait_gather(b)
            start_write(c, b)
        return carry

    lax.fori_loop(0, n_chunks // NBUF, outer, 0)

    for b in range(NBUF):
        wait_write(n_chunks - NBUF + b, b)


def kernel(inputs, weight):
    batch, length = inputs.shape
    total = batch * length
    flat_idx = inputs.reshape(total).astype(jnp.int32)
    mesh = plsc.VectorSubcoreMesh(core_axis_name="c", subcore_axis_name="s")
    scratch = ([pltpu.VMEM((CHUNK,), jnp.int32) for _ in range(NBUF)]
               + [pltpu.VMEM((CHUNK, HIDDEN), jnp.float32) for _ in range(NBUF)]
               + [pltpu.SemaphoreType.DMA for _ in range(2 * NBUF)])
    k = pl.kernel(
        _gather_body,
        mesh=mesh,
        out_type=jax.ShapeDtypeStruct((total, HIDDEN), jnp.float32),
        scratch_types=scratch,
        compiler_params=pltpu.CompilerParams(use_tc_tiling_on_sc=False),
    )
    out = k(flat_idx, weight)
    return out.reshape(batch, length, HIDDEN)
